# baseline (speedup 1.0000x reference)
import functools

import jax
import jax.numpy as jnp
import numpy as np
from jax import lax
from jax.experimental import pallas as pl
from jax.experimental.pallas import tpu as pltpu
from jax.experimental.pallas import tpu_sc as plsc

F32 = jnp.float32
BF16 = jnp.bfloat16
I32 = jnp.int32


def _fps_body(npoint, xs_ref, ys_ref, zs_ref, idx_ref, cx_ref, cy_ref, cz_ref):
    B, N = xs_ref.shape
    xs = xs_ref[...]
    ys = ys_ref[...]
    zs = zs_ref[...]
    iota = lax.broadcasted_iota(I32, (B, N), 1)
    iop = lax.broadcasted_iota(I32, (B, npoint), 1)

    def body(i, carry):
        dist, far = carry
        sel = iop == i
        idx_ref[...] = jnp.where(sel, jnp.broadcast_to(far, (B, npoint)),
                                 idx_ref[...])
        oh = (iota == far).astype(F32)
        cx = jnp.sum(xs * oh, axis=1, keepdims=True)
        cy = jnp.sum(ys * oh, axis=1, keepdims=True)
        cz = jnp.sum(zs * oh, axis=1, keepdims=True)
        cx_ref[...] = jnp.where(sel, jnp.broadcast_to(cx, (B, npoint)),
                                cx_ref[...])
        cy_ref[...] = jnp.where(sel, jnp.broadcast_to(cy, (B, npoint)),
                                cy_ref[...])
        cz_ref[...] = jnp.where(sel, jnp.broadcast_to(cz, (B, npoint)),
                                cz_ref[...])
        dx = xs - cx
        dy = ys - cy
        dz = zs - cz
        d = dx * dx
        d = d + dy * dy
        d = d + dz * dz
        dist = jnp.minimum(dist, d)
        m = jnp.max(dist, axis=1, keepdims=True)
        far = jnp.min(jnp.where(dist == m, iota, N), axis=1, keepdims=True)
        return dist, far

    idx_ref[...] = jnp.zeros((B, npoint), I32)
    cx_ref[...] = jnp.zeros((B, npoint), F32)
    cy_ref[...] = jnp.zeros((B, npoint), F32)
    cz_ref[...] = jnp.zeros((B, npoint), F32)
    dist0 = xs * 0.0 + 1e10
    far0 = jnp.min(iota, axis=1, keepdims=True)
    lax.fori_loop(0, npoint, body, (dist0, far0))


def _fps(xs, ys, zs, npoint):
    B, N = xs.shape
    outs = [jax.ShapeDtypeStruct((B, npoint), I32)] + \
           [jax.ShapeDtypeStruct((B, npoint), F32)] * 3
    return pl.pallas_call(
        functools.partial(_fps_body, npoint),
        out_shape=outs,
    )(xs, ys, zs)


def _bq_body(r2, K, N, ctr_ref, ctrb_ref, xyzTb_ref, xs_ref, ys_ref, zs_ref,
             gidx_ref, xn_ref, yn_ref, zn_ref, av_ref):
    S = ctr_ref.shape[1]
    ctr = ctr_ref[0]
    csq = jnp.sum(ctr * ctr, axis=1, keepdims=True)
    xs = xs_ref[0]
    ys = ys_ref[0]
    zs = zs_ref[0]
    xsq = xs * xs
    xsq = xsq + ys * ys
    xsq = xsq + zs * zs
    mm = jnp.dot(ctrb_ref[0], xyzTb_ref[0],
                 preferred_element_type=F32)
    d = -2.0 * mm
    d = d + csq
    d = d + xsq
    keep = d <= r2
    iota = lax.broadcasted_iota(I32, (S, N), 1)
    iok = lax.broadcasted_iota(I32, (S, K), 1)
    nval = jnp.int32(N)
    boff = pl.program_id(0) * nval

    xsb = jnp.broadcast_to(xs, (S, N))
    ysb = jnp.broadcast_to(ys, (S, N))
    zsb = jnp.broadcast_to(zs, (S, N))
    cx = ctr[:, 0:1]
    cy = ctr[:, 1:2]
    cz = ctr[:, 2:3]
    big = jnp.float32(-3.4e38)

    def body(k, _):
        avail = av_ref[...] != 0
        cand = jnp.where(avail, iota, nval)
        j = jnp.min(cand, axis=1, keepdims=True)
        jg = jnp.minimum(j, nval - 1) + boff
        first_g = gidx_ref[0][:, 0:1]
        jw = jnp.where(jnp.logical_and(j == nval, k > 0), first_g, jg)
        gidx_ref[0] = jnp.where(iok == k, jnp.broadcast_to(jw, (S, K)),
                                gidx_ref[0])
        m2 = iota == (jw - boff)
        xv = jnp.max(jnp.where(m2, xsb, big), axis=1, keepdims=True) - cx
        yv = jnp.max(jnp.where(m2, ysb, big), axis=1, keepdims=True) - cy
        zv = jnp.max(jnp.where(m2, zsb, big), axis=1, keepdims=True) - cz
        selk = iok == k
        xn_ref[0] = jnp.where(selk, jnp.broadcast_to(xv, (S, K)), xn_ref[0])
        yn_ref[0] = jnp.where(selk, jnp.broadcast_to(yv, (S, K)), yn_ref[0])
        zn_ref[0] = jnp.where(selk, jnp.broadcast_to(zv, (S, K)), zn_ref[0])
        av_ref[...] = jnp.where(iota == j, 0, av_ref[...])
        return 0

    gidx_ref[0] = jnp.zeros((S, K), I32)
    xn_ref[0] = jnp.zeros((S, K), F32)
    yn_ref[0] = jnp.zeros((S, K), F32)
    zn_ref[0] = jnp.zeros((S, K), F32)
    av_ref[...] = keep.astype(I32)
    lax.fori_loop(0, K, body, 0)


def _ballquery(ctr, ctr_bf, xyzT_bf, xs, ys, zs, r2, K):
    B, S, _ = ctr.shape
    N = xs.shape[1]
    return pl.pallas_call(
        functools.partial(_bq_body, np.float32(r2), K, N),
        grid=(B,),
        in_specs=[
            pl.BlockSpec((1, S, 8), lambda b: (b, 0, 0)),
            pl.BlockSpec((1, S, 8), lambda b: (b, 0, 0)),
            pl.BlockSpec((1, 8, N), lambda b: (b, 0, 0)),
            pl.BlockSpec((1, 1, N), lambda b: (b, 0, 0)),
            pl.BlockSpec((1, 1, N), lambda b: (b, 0, 0)),
            pl.BlockSpec((1, 1, N), lambda b: (b, 0, 0)),
        ],
        out_specs=[pl.BlockSpec((1, S, K), lambda b: (b, 0, 0))] * 4,
        out_shape=[jax.ShapeDtypeStruct((B, S, K), I32)] +
                  [jax.ShapeDtypeStruct((B, S, K), F32)] * 3,
        scratch_shapes=[pltpu.VMEM((S, N), I32)],
    )(ctr, ctr_bf, xyzT_bf, xs.reshape(B, 1, N), ys.reshape(B, 1, N),
      zs.reshape(B, 1, N))


def _sc_gather(table, gidx, D):
    R = gidx.shape[0]
    info = plsc.get_sparse_core_info()
    NC, NS = info.num_cores, info.num_subcores
    NW = NC * NS
    rpw = R // NW
    CH = min(rpw, max(8, 262144 // (D * 4)))
    nch = rpw // CH
    mesh = plsc.VectorSubcoreMesh(core_axis_name="c", subcore_axis_name="s")

    @functools.partial(
        pl.kernel, mesh=mesh,
        out_type=jax.ShapeDtypeStruct((R, D), F32),
        scratch_types=[
            pltpu.VMEM((CH,), I32),
            pltpu.VMEM((CH, D), F32),
            pltpu.SemaphoreType.DMA,
        ],
    )
    def k(table_hbm, idx_hbm, out_hbm, idx_v, rows_v, sem):
        wid = lax.axis_index("s") * NC + lax.axis_index("c")
        base = wid * rpw

        def step(ci, _):
            off = base + ci * CH
            pltpu.sync_copy(idx_hbm.at[pl.ds(off, CH)], idx_v)
            pltpu.async_copy(table_hbm.at[idx_v], rows_v, sem).wait()
            pltpu.sync_copy(rows_v, out_hbm.at[pl.ds(off, CH)])
            return 0

        lax.fori_loop(0, nch, step, 0)

    return k(table, gidx)


def _stats_init_and_acc(st_ref, y):
    @pl.when(pl.program_id(0) == 0)
    def _():
        st_ref[...] = jnp.zeros_like(st_ref)
    st_ref[0:1, :] += jnp.sum(y, axis=0, keepdims=True)
    st_ref[1:2, :] += jnp.sum(y * y, axis=0, keepdims=True)


def _scale_shift(st_ref, gb_ref, n):
    st = st_ref[...]
    mean = st[0:1, :] / n
    var = st[1:2, :] / n - mean * mean
    g = gb_ref[0:1, :]
    be = gb_ref[1:2, :]
    scale = g / jnp.sqrt(var + 1e-5)
    shift = be - mean * scale
    return scale, shift


def _l1_stats_body(Xn_ref, wa_ref, Gp_ref, wb_ref, b_ref, st_ref):
    y = jnp.dot(Xn_ref[...].astype(BF16), wa_ref[...], preferred_element_type=F32)
    if Gp_ref is not None:
        y = y + jnp.dot(Gp_ref[...].astype(BF16), wb_ref[...],
                        preferred_element_type=F32)
    y = y + b_ref[...]
    _stats_init_and_acc(st_ref, y)


def _l1_layer_body(n, Xn_ref, wa_ref, Gp_ref, wb_ref, b_ref,
                   st_ref, gb_ref, w2_ref, b2_ref, y2_ref, st2_ref):
    y = jnp.dot(Xn_ref[...].astype(BF16), wa_ref[...], preferred_element_type=F32)
    if Gp_ref is not None:
        y = y + jnp.dot(Gp_ref[...].astype(BF16), wb_ref[...],
                        preferred_element_type=F32)
    y = y + b_ref[...]
    scale, shift = _scale_shift(st_ref, gb_ref, n)
    xn = jnp.maximum(y * scale + shift, 0.0)
    y2 = jnp.dot(xn.astype(BF16), w2_ref[...], preferred_element_type=F32)
    y2 = y2 + b2_ref[...]
    y2_ref[...] = y2
    _stats_init_and_acc(st2_ref, y2)


def _mm_stats_body(x_ref, w_ref, b_ref, y_ref, st_ref):
    y = jnp.dot(x_ref[...].astype(BF16), w_ref[...], preferred_element_type=F32)
    y = y + b_ref[...]
    y_ref[...] = y
    _stats_init_and_acc(st_ref, y)


def _layer_body(n, x_ref, st_ref, gb_ref, w_ref, b_ref, y_ref, st2_ref):
    scale, shift = _scale_shift(st_ref, gb_ref, n)
    xn = jnp.maximum(x_ref[...] * scale + shift, 0.0)
    y = jnp.dot(xn.astype(BF16), w_ref[...], preferred_element_type=F32)
    y = y + b_ref[...]
    y_ref[...] = y
    _stats_init_and_acc(st2_ref, y)


def _pool_body(n, y_ref, st_ref, gb_ref, o_ref):
    scale, shift = _scale_shift(st_ref, gb_ref, n)
    xn = jnp.maximum(y_ref[...] * scale[None] + shift[None], 0.0)
    o_ref[...] = jnp.max(xn, axis=1)


def _full_spec(shape):
    nd = len(shape)
    return pl.BlockSpec(shape, lambda i: (0,) * nd)


def _row_spec(rb, cols):
    return pl.BlockSpec((rb, cols), lambda i: (i, 0))


def _st_shape(C):
    return jax.ShapeDtypeStruct((8, C), F32)


def _l1_stats(K, Xn, wa, Gp, wb, b, rb):
    R, Dx = Xn.shape
    C = wa.shape[1]
    args = [Xn, wa] + ([Gp, wb] if Gp is not None else []) + [b]
    specs = [_row_spec(rb, Dx), _full_spec(wa.shape)] + \
            ([_row_spec(rb, Gp.shape[1]), _full_spec(wb.shape)]
             if Gp is not None else []) + [_full_spec(b.shape)]

    def body(*refs):
        if Gp is not None:
            _l1_stats_body(*refs)
        else:
            _l1_stats_body(refs[0], refs[1], None, None, refs[2], refs[3])

    return pl.pallas_call(
        body, grid=(R // rb,), in_specs=specs,
        out_specs=_full_spec((8, C)), out_shape=_st_shape(C),
    )(*args)


def _l1_layer(K, n, Xn, wa, Gp, wb, b, st, gb, w2, b2, rb):
    R, Dx = Xn.shape
    C2 = w2.shape[1]
    args = [Xn, wa] + ([Gp, wb] if Gp is not None else []) + \
           [b, st, gb, w2, b2]
    specs = [_row_spec(rb, Dx), _full_spec(wa.shape)] + \
            ([_row_spec(rb, Gp.shape[1]), _full_spec(wb.shape)]
             if Gp is not None else []) + \
            [_full_spec(b.shape), _full_spec(st.shape), _full_spec(gb.shape),
             _full_spec(w2.shape), _full_spec(b2.shape)]

    def body(*refs):
        if Gp is not None:
            _l1_layer_body(n, *refs)
        else:
            r = refs
            _l1_layer_body(n, r[0], r[1], None, None, r[2], r[3],
                           r[4], r[5], r[6], r[7], r[8])

    return pl.pallas_call(
        body, grid=(R // rb,), in_specs=specs,
        out_specs=[_row_spec(rb, C2), _full_spec((8, C2))],
        out_shape=[jax.ShapeDtypeStruct((R, C2), F32), _st_shape(C2)],
    )(*args)


def _mm_stats(x, w, b, rb):
    R = x.shape[0]
    C = w.shape[1]
    return pl.pallas_call(
        _mm_stats_body, grid=(R // rb,),
        in_specs=[_row_spec(rb, x.shape[1]), _full_spec(w.shape),
                  _full_spec(b.shape)],
        out_specs=[_row_spec(rb, C), _full_spec((8, C))],
        out_shape=[jax.ShapeDtypeStruct((R, C), F32), _st_shape(C)],
    )(x, w, b)


def _layer(n, x, st, gb, w, b, rb):
    R = x.shape[0]
    C = w.shape[1]
    return pl.pallas_call(
        functools.partial(_layer_body, n), grid=(R // rb,),
        in_specs=[_row_spec(rb, x.shape[1]), _full_spec(st.shape),
                  _full_spec(gb.shape), _full_spec(w.shape),
                  _full_spec(b.shape)],
        out_specs=[_row_spec(rb, C), _full_spec((8, C))],
        out_shape=[jax.ShapeDtypeStruct((R, C), F32), _st_shape(C)],
    )(x, st, gb, w, b)


def _pool(n, y, st, gb, sb):
    NS_, K, C = y.shape
    return pl.pallas_call(
        functools.partial(_pool_body, n), grid=(NS_ // sb,),
        in_specs=[pl.BlockSpec((sb, K, C), lambda i: (i, 0, 0)),
                  _full_spec(st.shape), _full_spec(gb.shape)],
        out_specs=pl.BlockSpec((sb, C), lambda i: (i, 0)),
        out_shape=jax.ShapeDtypeStruct((NS_, C), F32),
    )(y, st, gb)


def _prep_layers(layers):
    out = []
    for (W, b, g, be) in layers:
        C = W.shape[0]
        wt = jnp.transpose(W).astype(BF16)
        bb = b.reshape(1, C)
        gb = jnp.concatenate([g.reshape(1, C), be.reshape(1, C),
                              jnp.zeros((6, C), F32)], axis=0)
        out.append((wt, bb, gb))
    return out


def _pad8(x3):
    pad = x3.shape[:-1] + (8 - x3.shape[-1],)
    return jnp.concatenate([x3, jnp.zeros(pad, x3.dtype)], axis=-1)


def _sa_grouped(xs, ys, zs, xyzT, pts, npoint, r2, K, layers, rb, rb3, sb_pool):
    B, N = xs.shape
    _, cx, cy, cz = _fps(xs, ys, zs, npoint)
    ctr = jnp.stack([cx, cy, cz], axis=-1)
    ctr8 = _pad8(ctr)
    txyz = _pad8(jnp.transpose(xyzT, (0, 2, 1)))
    gidx, xn, yn, zn = _ballquery(ctr8, ctr8.astype(BF16),
                                  jnp.transpose(txyz.astype(BF16), (0, 2, 1)),
                                  xs, ys, zs, r2, K)
    Xn = _pad8(jnp.stack([xn, yn, zn], axis=-1)).reshape(-1, 8)
    if pts is not None:
        Cp = pts.shape[-1]
        Gp = _sc_gather(pts.reshape(B * N, Cp), gidx.reshape(-1), Cp)
    else:
        Gp = None

    (w1, b1, gb1), (w2, b2, gb2), (w3, b3, gb3) = _prep_layers(layers)
    C1 = w1.shape[1]
    wa = w1[0:3, :]
    wa8 = jnp.concatenate([wa, jnp.zeros((5, C1), BF16)], axis=0)
    wb = w1[3:, :] if pts is not None else None

    R = B * npoint * K
    n = float(R)
    st1 = _l1_stats(K, Xn, wa8, Gp, wb, b1, rb)
    y2, st2 = _l1_layer(K, n, Xn, wa8, Gp, wb, b1, st1, gb1, w2, b2, rb)
    y3, st3 = _layer(n, y2, st2, gb2, w3, b3, rb3)
    C3 = w3.shape[1]
    out = _pool(n, y3.reshape(B * npoint, K, C3), st3, gb3, sb_pool)
    return ctr, cx, cy, cz, out.reshape(B, npoint, C3)


def kernel(xyz, params):
    B, _, N = xyz.shape
    xs = xyz[:, 0, :]
    ys = xyz[:, 1, :]
    zs = xyz[:, 2, :]

    ctr1, c1x, c1y, c1z, l1_points = _sa_grouped(
        xs, ys, zs, xyz, None, 512, 0.0176 ** 2, 32, params['sa1'],
        rb=8192, rb3=8192, sb_pool=128)
    del ctr1

    xyzT2 = jnp.stack([c1x, c1y, c1z], axis=1)
    ctr2, c2x, c2y, c2z, l2_points = _sa_grouped(
        c1x, c1y, c1z, xyzT2, l1_points, 128, 2.3466 ** 2, 64, params['sa2'],
        rb=4096, rb3=2048, sb_pool=32)
    del c2x, c2y, c2z

    x3 = jnp.concatenate([ctr2, l2_points], axis=-1).reshape(B * 128, 1027)
    (w1, b1, gb1), (w2, b2, gb2) = _prep_layers(params['sa3'])
    n3 = float(B * 128)
    y1, st1 = _mm_stats(x3, w1, b1, rb=512)
    y2, st2 = _layer(n3, y1, st1, gb1, w2, b2, rb=512)
    out = _pool(n3, y2.reshape(B, 128, 1024), st2, gb2, sb=B)
    return out.reshape(B, 1024)


# speedup vs baseline: 1.2637x; 1.2637x over previous
import functools

import jax
import jax.numpy as jnp
import numpy as np
from jax import lax
from jax.experimental import pallas as pl
from jax.experimental.pallas import tpu as pltpu
from jax.experimental.pallas import tpu_sc as plsc

F32 = jnp.float32
BF16 = jnp.bfloat16
I32 = jnp.int32


def _fps_body(npoint, xs_ref, ys_ref, zs_ref, idx_ref, cx_ref, cy_ref, cz_ref):
    B, N = xs_ref.shape
    xs = xs_ref[...]
    ys = ys_ref[...]
    zs = zs_ref[...]
    iota = lax.broadcasted_iota(I32, (B, N), 1)
    iop = lax.broadcasted_iota(I32, (B, npoint), 1)

    def body(i, carry):
        dist, far = carry
        sel = iop == i
        idx_ref[...] = jnp.where(sel, jnp.broadcast_to(far, (B, npoint)),
                                 idx_ref[...])
        oh = (iota == far).astype(F32)
        cx = jnp.sum(xs * oh, axis=1, keepdims=True)
        cy = jnp.sum(ys * oh, axis=1, keepdims=True)
        cz = jnp.sum(zs * oh, axis=1, keepdims=True)
        cx_ref[...] = jnp.where(sel, jnp.broadcast_to(cx, (B, npoint)),
                                cx_ref[...])
        cy_ref[...] = jnp.where(sel, jnp.broadcast_to(cy, (B, npoint)),
                                cy_ref[...])
        cz_ref[...] = jnp.where(sel, jnp.broadcast_to(cz, (B, npoint)),
                                cz_ref[...])
        dx = xs - cx
        dy = ys - cy
        dz = zs - cz
        d = dx * dx
        d = d + dy * dy
        d = d + dz * dz
        dist = jnp.minimum(dist, d)
        m = jnp.max(dist, axis=1, keepdims=True)
        far = jnp.min(jnp.where(dist == m, iota, N), axis=1, keepdims=True)
        return dist, far

    idx_ref[...] = jnp.zeros((B, npoint), I32)
    cx_ref[...] = jnp.zeros((B, npoint), F32)
    cy_ref[...] = jnp.zeros((B, npoint), F32)
    cz_ref[...] = jnp.zeros((B, npoint), F32)
    dist0 = xs * 0.0 + 1e10
    far0 = jnp.min(iota, axis=1, keepdims=True)
    lax.fori_loop(0, npoint, body, (dist0, far0))


def _fps(xs, ys, zs, npoint):
    B, N = xs.shape
    outs = [jax.ShapeDtypeStruct((B, npoint), I32)] + \
           [jax.ShapeDtypeStruct((B, npoint), F32)] * 3
    return pl.pallas_call(
        functools.partial(_fps_body, npoint),
        out_shape=outs,
    )(xs, ys, zs)


def _bq_body(r2, K, N, ctr_ref, ctrb_ref, xyzTb_ref, xs_ref, ys_ref, zs_ref,
             gidx_ref, av_ref):
    S = ctr_ref.shape[1]
    ctr = ctr_ref[0]
    csq = jnp.sum(ctr * ctr, axis=1, keepdims=True)
    xs = xs_ref[0]
    ys = ys_ref[0]
    zs = zs_ref[0]
    xsq = xs * xs
    xsq = xsq + ys * ys
    xsq = xsq + zs * zs
    mm = jnp.dot(ctrb_ref[0], xyzTb_ref[0],
                 preferred_element_type=F32)
    d = -2.0 * mm
    d = d + csq
    d = d + xsq
    keep = d <= r2
    iota = lax.broadcasted_iota(I32, (S, N), 1)
    iok = lax.broadcasted_iota(I32, (S, K), 1)
    nval = jnp.int32(N)
    boff = pl.program_id(0) * nval

    def body(k, _):
        avail = av_ref[...] != 0
        cand = jnp.where(avail, iota, nval)
        j = jnp.min(cand, axis=1, keepdims=True)
        jg = jnp.minimum(j, nval - 1) + boff
        first_g = gidx_ref[0][:, 0:1]
        jw = jnp.where(jnp.logical_and(j == nval, k > 0), first_g, jg)
        gidx_ref[0] = jnp.where(iok == k, jnp.broadcast_to(jw, (S, K)),
                                gidx_ref[0])
        av_ref[...] = jnp.where(iota == j, 0, av_ref[...])
        return 0

    gidx_ref[0] = jnp.zeros((S, K), I32)
    av_ref[...] = keep.astype(I32)
    lax.fori_loop(0, K, body, 0)


def _ballquery(ctr, ctr_bf, xyzT_bf, xs, ys, zs, r2, K):
    B, S, _ = ctr.shape
    N = xs.shape[1]
    return pl.pallas_call(
        functools.partial(_bq_body, np.float32(r2), K, N),
        grid=(B,),
        in_specs=[
            pl.BlockSpec((1, S, 8), lambda b: (b, 0, 0)),
            pl.BlockSpec((1, S, 8), lambda b: (b, 0, 0)),
            pl.BlockSpec((1, 8, N), lambda b: (b, 0, 0)),
            pl.BlockSpec((1, 1, N), lambda b: (b, 0, 0)),
            pl.BlockSpec((1, 1, N), lambda b: (b, 0, 0)),
            pl.BlockSpec((1, 1, N), lambda b: (b, 0, 0)),
        ],
        out_specs=pl.BlockSpec((1, S, K), lambda b: (b, 0, 0)),
        out_shape=jax.ShapeDtypeStruct((B, S, K), I32),
        scratch_shapes=[pltpu.VMEM((S, N), I32)],
    )(ctr, ctr_bf, xyzT_bf, xs.reshape(B, 1, N), ys.reshape(B, 1, N),
      zs.reshape(B, 1, N))


def _sc_gather(table, gidx, D):
    R = gidx.shape[0]
    info = plsc.get_sparse_core_info()
    NC, NS = info.num_cores, info.num_subcores
    NW = NC * NS
    rpw = R // NW
    CH = 1 << (min(rpw, max(8, 262144 // (D * 4))).bit_length() - 1)
    nch = rpw // CH
    mesh = plsc.VectorSubcoreMesh(core_axis_name="c", subcore_axis_name="s")

    @functools.partial(
        pl.kernel, mesh=mesh,
        out_type=jax.ShapeDtypeStruct((R, D), F32),
        scratch_types=[
            pltpu.VMEM((CH,), I32),
            pltpu.VMEM((CH, D), F32),
            pltpu.SemaphoreType.DMA,
        ],
    )
    def k(table_hbm, idx_hbm, out_hbm, idx_v, rows_v, sem):
        wid = lax.axis_index("s") * NC + lax.axis_index("c")
        base = wid * rpw

        def step(ci, _):
            off = base + ci * CH
            pltpu.sync_copy(idx_hbm.at[pl.ds(off, CH)], idx_v)
            pltpu.async_copy(table_hbm.at[idx_v], rows_v, sem).wait()
            pltpu.sync_copy(rows_v, out_hbm.at[pl.ds(off, CH)])
            return 0

        lax.fori_loop(0, nch, step, 0)

    return k(table, gidx)


def _stats_init_and_acc(st_ref, y):
    @pl.when(pl.program_id(0) == 0)
    def _():
        st_ref[...] = jnp.zeros_like(st_ref)
    st_ref[0:1, :] += jnp.sum(y, axis=0, keepdims=True)
    st_ref[1:2, :] += jnp.sum(y * y, axis=0, keepdims=True)


def _scale_shift(st_ref, gb_ref, n):
    st = st_ref[...]
    mean = st[0:1, :] / n
    var = st[1:2, :] / n - mean * mean
    g = gb_ref[0:1, :]
    be = gb_ref[1:2, :]
    scale = g / jnp.sqrt(var + 1e-5)
    shift = be - mean * scale
    return scale, shift


def _l1_pre(K, xoff, Cp, Gc_ref, c_ref, wa_ref, wb_ref, b_ref):
    Gc = Gc_ref[...]
    c8 = c_ref[...]
    sb = c8.shape[0]
    gx = Gc[:, xoff:xoff + 8]
    gx = (gx.reshape(sb, K, 8) - c8[:, None, :]).reshape(sb * K, 8)
    y = jnp.dot(gx.astype(BF16), wa_ref[...], preferred_element_type=F32)
    if Cp:
        y = y + jnp.dot(Gc[:, 0:Cp].astype(BF16), wb_ref[...],
                        preferred_element_type=F32)
    y = y + b_ref[...]
    return y


def _l1_stats_body(K, xoff, Cp, Gc_ref, c_ref, wa_ref, wb_ref, b_ref, st_ref):
    y = _l1_pre(K, xoff, Cp, Gc_ref, c_ref, wa_ref, wb_ref, b_ref)
    _stats_init_and_acc(st_ref, y)


def _l1_layer_body(K, xoff, Cp, n, Gc_ref, c_ref, wa_ref, wb_ref, b_ref,
                   st_ref, gb_ref, w2_ref, b2_ref, y2_ref, st2_ref):
    y = _l1_pre(K, xoff, Cp, Gc_ref, c_ref, wa_ref, wb_ref, b_ref)
    scale, shift = _scale_shift(st_ref, gb_ref, n)
    xn = jnp.maximum(y * scale + shift, 0.0)
    y2 = jnp.dot(xn.astype(BF16), w2_ref[...], preferred_element_type=F32)
    y2 = y2 + b2_ref[...]
    y2_ref[...] = y2
    _stats_init_and_acc(st2_ref, y2)


def _mm_stats_body(x_ref, w_ref, b_ref, y_ref, st_ref):
    y = jnp.dot(x_ref[...].astype(BF16), w_ref[...], preferred_element_type=F32)
    y = y + b_ref[...]
    y_ref[...] = y
    _stats_init_and_acc(st_ref, y)


def _layer_body(n, x_ref, st_ref, gb_ref, w_ref, b_ref, y_ref, st2_ref):
    scale, shift = _scale_shift(st_ref, gb_ref, n)
    xn = jnp.maximum(x_ref[...] * scale + shift, 0.0)
    y = jnp.dot(xn.astype(BF16), w_ref[...], preferred_element_type=F32)
    y = y + b_ref[...]
    y_ref[...] = y
    _stats_init_and_acc(st2_ref, y)


def _pool_body(n, y_ref, st_ref, gb_ref, o_ref):
    scale, shift = _scale_shift(st_ref, gb_ref, n)
    xn = jnp.maximum(y_ref[...] * scale[None] + shift[None], 0.0)
    o_ref[...] = jnp.max(xn, axis=1)


def _full_spec(shape):
    nd = len(shape)
    return pl.BlockSpec(shape, lambda i: (0,) * nd)


def _row_spec(rb, cols):
    return pl.BlockSpec((rb, cols), lambda i: (i, 0))


def _st_shape(C):
    return jax.ShapeDtypeStruct((8, C), F32)


def _wb_or_dummy(wb, wa):
    return wb if wb is not None else wa


def _l1_stats(K, xoff, Cp, Gc, c8, wa, wb, b, rb):
    R, D = Gc.shape
    C = wa.shape[1]
    sb = rb // K
    specs = [_row_spec(rb, D), pl.BlockSpec((sb, 8), lambda i: (i, 0)),
             _full_spec(wa.shape), _full_spec(_wb_or_dummy(wb, wa).shape),
             _full_spec(b.shape)]
    return pl.pallas_call(
        functools.partial(_l1_stats_body, K, xoff, Cp),
        grid=(R // rb,), in_specs=specs,
        out_specs=_full_spec((8, C)), out_shape=_st_shape(C),
    )(Gc, c8, wa, _wb_or_dummy(wb, wa), b)


def _l1_layer(K, xoff, Cp, n, Gc, c8, wa, wb, b, st, gb, w2, b2, rb):
    R, D = Gc.shape
    C2 = w2.shape[1]
    sb = rb // K
    specs = [_row_spec(rb, D), pl.BlockSpec((sb, 8), lambda i: (i, 0)),
             _full_spec(wa.shape), _full_spec(_wb_or_dummy(wb, wa).shape),
             _full_spec(b.shape), _full_spec(st.shape), _full_spec(gb.shape),
             _full_spec(w2.shape), _full_spec(b2.shape)]
    return pl.pallas_call(
        functools.partial(_l1_layer_body, K, xoff, Cp, n),
        grid=(R // rb,), in_specs=specs,
        out_specs=[_row_spec(rb, C2), _full_spec((8, C2))],
        out_shape=[jax.ShapeDtypeStruct((R, C2), F32), _st_shape(C2)],
    )(Gc, c8, wa, _wb_or_dummy(wb, wa), b, st, gb, w2, b2)


def _mm_stats(x, w, b, rb):
    R = x.shape[0]
    C = w.shape[1]
    return pl.pallas_call(
        _mm_stats_body, grid=(R // rb,),
        in_specs=[_row_spec(rb, x.shape[1]), _full_spec(w.shape),
                  _full_spec(b.shape)],
        out_specs=[_row_spec(rb, C), _full_spec((8, C))],
        out_shape=[jax.ShapeDtypeStruct((R, C), F32), _st_shape(C)],
    )(x, w, b)


def _layer(n, x, st, gb, w, b, rb):
    R = x.shape[0]
    C = w.shape[1]
    return pl.pallas_call(
        functools.partial(_layer_body, n), grid=(R // rb,),
        in_specs=[_row_spec(rb, x.shape[1]), _full_spec(st.shape),
                  _full_spec(gb.shape), _full_spec(w.shape),
                  _full_spec(b.shape)],
        out_specs=[_row_spec(rb, C), _full_spec((8, C))],
        out_shape=[jax.ShapeDtypeStruct((R, C), F32), _st_shape(C)],
    )(x, st, gb, w, b)


def _pool(n, y, st, gb, sb):
    NS_, K, C = y.shape
    return pl.pallas_call(
        functools.partial(_pool_body, n), grid=(NS_ // sb,),
        in_specs=[pl.BlockSpec((sb, K, C), lambda i: (i, 0, 0)),
                  _full_spec(st.shape), _full_spec(gb.shape)],
        out_specs=pl.BlockSpec((sb, C), lambda i: (i, 0)),
        out_shape=jax.ShapeDtypeStruct((NS_, C), F32),
    )(y, st, gb)


def _prep_layers(layers):
    out = []
    for (W, b, g, be) in layers:
        C = W.shape[0]
        wt = jnp.transpose(W).astype(BF16)
        bb = b.reshape(1, C)
        gb = jnp.concatenate([g.reshape(1, C), be.reshape(1, C),
                              jnp.zeros((6, C), F32)], axis=0)
        out.append((wt, bb, gb))
    return out


def _pad8(x3):
    pad = x3.shape[:-1] + (8 - x3.shape[-1],)
    return jnp.concatenate([x3, jnp.zeros(pad, x3.dtype)], axis=-1)


def _sa_grouped(xs, ys, zs, xyzT, pts, npoint, r2, K, layers, rb, rb3, sb_pool):
    B, N = xs.shape
    _, cx, cy, cz = _fps(xs, ys, zs, npoint)
    ctr = jnp.stack([cx, cy, cz], axis=-1)
    ctr8 = _pad8(ctr)
    txyz = _pad8(jnp.transpose(xyzT, (0, 2, 1)))
    gidx = _ballquery(ctr8, ctr8.astype(BF16),
                      jnp.transpose(txyz.astype(BF16), (0, 2, 1)),
                      xs, ys, zs, r2, K)
    gflat = gidx.reshape(-1)
    if pts is not None:
        Cp = pts.shape[-1]
        xoff = Cp
        D = ((Cp + 8 + 127) // 128) * 128
        tbl = jnp.concatenate(
            [pts.reshape(B * N, Cp), txyz.reshape(B * N, 8),
             jnp.zeros((B * N, D - Cp - 8), F32)], axis=-1)
    else:
        Cp = 0
        xoff = 0
        D = 128
        tbl = jnp.concatenate(
            [txyz.reshape(B * N, 8), jnp.zeros((B * N, 120), F32)], axis=-1)
    Gc = _sc_gather(tbl, gflat, D)
    c8 = ctr8.reshape(B * npoint, 8)

    (w1, b1, gb1), (w2, b2, gb2), (w3, b3, gb3) = _prep_layers(layers)
    C1 = w1.shape[1]
    wa = w1[0:3, :]
    wa8 = jnp.concatenate([wa, jnp.zeros((5, C1), BF16)], axis=0)
    wb = w1[3:, :] if pts is not None else None

    R = B * npoint * K
    n = float(R)
    st1 = _l1_stats(K, xoff, Cp, Gc, c8, wa8, wb, b1, rb)
    y2, st2 = _l1_layer(K, xoff, Cp, n, Gc, c8, wa8, wb, b1, st1, gb1,
                        w2, b2, rb)
    y3, st3 = _layer(n, y2, st2, gb2, w3, b3, rb3)
    C3 = w3.shape[1]
    out = _pool(n, y3.reshape(B * npoint, K, C3), st3, gb3, sb_pool)
    return ctr, cx, cy, cz, out.reshape(B, npoint, C3)


def kernel(xyz, params):
    B, _, N = xyz.shape
    xs = xyz[:, 0, :]
    ys = xyz[:, 1, :]
    zs = xyz[:, 2, :]

    ctr1, c1x, c1y, c1z, l1_points = _sa_grouped(
        xs, ys, zs, xyz, None, 512, 0.0176 ** 2, 32, params['sa1'],
        rb=8192, rb3=8192, sb_pool=128)
    del ctr1

    xyzT2 = jnp.stack([c1x, c1y, c1z], axis=1)
    ctr2, c2x, c2y, c2z, l2_points = _sa_grouped(
        c1x, c1y, c1z, xyzT2, l1_points, 128, 2.3466 ** 2, 64, params['sa2'],
        rb=4096, rb3=2048, sb_pool=32)
    del c2x, c2y, c2z

    x3 = jnp.concatenate([ctr2, l2_points], axis=-1).reshape(B * 128, 1027)
    (w1, b1, gb1), (w2, b2, gb2) = _prep_layers(params['sa3'])
    n3 = float(B * 128)
    y1, st1 = _mm_stats(x3, w1, b1, rb=512)
    y2, st2 = _layer(n3, y1, st1, gb1, w2, b2, rb=512)
    out = _pool(n3, y2.reshape(B, 128, 1024), st2, gb2, sb=B)
    return out.reshape(B, 1024)
